# Initial kernel scaffold; baseline (speedup 1.0000x reference)
import jax
import jax.numpy as jnp
from jax.experimental import pallas as pl


def kernel(user_emb, item_emb, W0, b0, W1, b1, rows, cols):
    raise NotImplementedError("write your pallas kernel here")


# speedup vs baseline: 1.1044x; 1.1044x over previous
import functools

import jax
import jax.numpy as jnp
from jax.experimental import pallas as pl
from jax.experimental.pallas import tpu as pltpu

N_U = 2048
N_I = 2048
D = 64
EPS = 1e-7


def _mm(A, B):
    return jax.lax.dot_general(A, B, (((1,), (0,)), ((), ())),
                               preferred_element_type=jnp.float32)


def _dhcf_body(H_ref, HT_ref, u_ref, i_ref, W0_ref, b0_ref, W1_ref, b1_ref,
               u1_ref, u2_ref, i1_ref, i2_ref):
    H = H_ref[...]
    HT = HT_ref[...]

    rs = jnp.sum(H, axis=1, keepdims=True)
    cs = jnp.sum(HT, axis=1, keepdims=True)
    Grs = _mm(H, _mm(HT, rs))
    Gcs = _mm(HT, _mm(H, cs))

    dv_u = jax.lax.rsqrt(rs + Grs + EPS)
    de1_u = 1.0 / (cs + EPS)
    de2_u = 1.0 / (Gcs + EPS)
    dv_i = jax.lax.rsqrt(cs + Gcs + EPS)
    de1_i = 1.0 / (rs + EPS)
    de2_i = 1.0 / (Grs + EPS)

    def side(X, A, AT, dv, de1, de2):
        Y = dv * X
        a = _mm(AT, Y)
        b2 = _mm(AT, _mm(A, a))
        t1 = de1 * a
        t2 = de2 * b2
        z = _mm(A, t1 + _mm(AT, _mm(A, t2)))
        return dv * z + X

    U = u_ref[...]
    I = i_ref[...]
    W0 = W0_ref[...]
    b0 = b0_ref[...]
    W1 = W1_ref[...]
    b1 = b1_ref[...]

    U = _mm(side(U, H, HT, dv_u, de1_u, de2_u), W0) + b0
    I = _mm(side(I, HT, H, dv_i, de1_i, de2_i), W0) + b0
    u1_ref[...] = U
    i1_ref[...] = I
    U = _mm(side(U, H, HT, dv_u, de1_u, de2_u), W1) + b1
    I = _mm(side(I, HT, H, dv_i, de1_i, de2_i), W1) + b1
    u2_ref[...] = U
    i2_ref[...] = I


@functools.partial(jax.jit, static_argnames=("interpret",))
def _dhcf_tc(H, HT, user_emb, item_emb, W0, b0, W1, b1, interpret=False):
    out = jax.ShapeDtypeStruct((N_U, D), jnp.float32)
    return pl.pallas_call(
        _dhcf_body,
        out_shape=(out, out, out, out),
        interpret=interpret,
    )(H, HT, user_emb, item_emb, W0, b0.reshape(1, D), W1, b1.reshape(1, D))


def kernel(user_emb, item_emb, W0, b0, W1, b1, rows, cols):
    H = jnp.zeros((N_U, N_I), jnp.float32).at[rows, cols].add(1.0)
    HT = jnp.zeros((N_I, N_U), jnp.float32).at[cols, rows].add(1.0)
    u1, u2, i1, i2 = _dhcf_tc(H, HT, user_emb, item_emb, W0, b0, W1, b1)
    U_out = jnp.concatenate([user_emb, u1, u2], axis=1)
    I_out = jnp.concatenate([item_emb, i1, i2], axis=1)
    return (U_out, I_out)


# speedup vs baseline: 1.2501x; 1.1320x over previous
import functools

import jax
import jax.numpy as jnp
from jax import lax
from jax.experimental import pallas as pl
from jax.experimental.pallas import tpu as pltpu
from jax.experimental.pallas import tpu_sc as plsc

N_U = 2048
N_I = 2048
D = 64
EPS = 1e-7

N_EDGE = 32768
N_TILES = 16
EPT = N_EDGE // N_TILES
CHUNK = (N_U // 4) * N_I
ZBLK = CHUNK // N_TILES
LANES = 16


def _sc_build_body(rows_h, cols_h, zeros_h, ones_h, hf_h, htf_h,
                   r_v, c_v, flat_v, idx_v, ones_v, acc):
    cid = lax.axis_index("c")
    sid = lax.axis_index("s")
    is_h = cid == 0

    if True:
        base = sid * EPT
        pltpu.sync_copy(rows_h.at[pl.ds(base, EPT)], r_v)
        pltpu.sync_copy(cols_h.at[pl.ds(base, EPT)], c_v)
        pltpu.sync_copy(ones_h, ones_v)

        def flat_body(i, _):
            rr = r_v[pl.ds(i * LANES, LANES)]
            cc = c_v[pl.ds(i * LANES, LANES)]
            maj = jnp.where(is_h, rr, cc)
            mnr = jnp.where(is_h, cc, rr)
            flat_v[pl.ds(i * LANES, LANES)] = maj * N_I + mnr
            return 0

        lax.fori_loop(0, EPT // LANES, flat_body, 0)

        for p in range(N_U * N_I // CHUNK):
            pltpu.sync_copy(zeros_h, acc.at[pl.ds(sid * ZBLK, ZBLK)])
            plsc.subcore_barrier()

            def idx_body(i, _):
                fl = flat_v[pl.ds(i * LANES, LANES)]
                loc = fl - p * CHUNK
                valid = (loc >= 0) & (loc < CHUNK)
                idx_v[pl.ds(i * LANES, LANES)] = jnp.where(valid, loc, CHUNK)
                return 0

            lax.fori_loop(0, EPT // LANES, idx_body, 0)
            pltpu.sync_copy(ones_v, acc.at[idx_v], add=True)
            plsc.subcore_barrier()

            @pl.when(is_h)
            def _():
                pltpu.sync_copy(acc.at[pl.ds(sid * ZBLK, ZBLK)],
                                hf_h.at[pl.ds(p * CHUNK + sid * ZBLK, ZBLK)])

            @pl.when(jnp.logical_not(is_h))
            def _():
                pltpu.sync_copy(acc.at[pl.ds(sid * ZBLK, ZBLK)],
                                htf_h.at[pl.ds(p * CHUNK + sid * ZBLK, ZBLK)])

            plsc.subcore_barrier()


@jax.jit
def _sc_build(rows, cols):
    zeros = jnp.zeros((ZBLK,), jnp.float32)
    ones = jnp.ones((EPT,), jnp.float32)
    out = jax.ShapeDtypeStruct((N_U * N_I,), jnp.float32)
    f = pl.kernel(
        _sc_build_body,
        out_type=(out, out),
        mesh=plsc.VectorSubcoreMesh(core_axis_name="c", subcore_axis_name="s",
                                    num_cores=2, num_subcores=16),
        scratch_types=[
            pltpu.VMEM((EPT,), jnp.int32),
            pltpu.VMEM((EPT,), jnp.int32),
            pltpu.VMEM((EPT,), jnp.int32),
            pltpu.VMEM((EPT,), jnp.int32),
            pltpu.VMEM((EPT,), jnp.float32),
            pltpu.VMEM_SHARED((CHUNK + LANES,), jnp.float32),
        ],
    )
    return f(rows, cols, zeros, ones)


def _mm(A, B):
    return jax.lax.dot_general(A, B, (((1,), (0,)), ((), ())),
                               preferred_element_type=jnp.float32)


def _dhcf_body(H_ref, HT_ref, u_ref, i_ref, W0_ref, b0_ref, W1_ref, b1_ref,
               u1_ref, u2_ref, i1_ref, i2_ref):
    H = H_ref[...]
    HT = HT_ref[...]

    rs = jnp.sum(H, axis=1, keepdims=True)
    cs = jnp.sum(HT, axis=1, keepdims=True)
    p0 = _mm(HT, rs)
    gq = _mm(H, jnp.concatenate([p0, cs], axis=1))
    Grs = gq[:, 0:1]
    Gcs = _mm(HT, gq[:, 1:2])

    dv_u = jax.lax.rsqrt(rs + Grs + EPS)
    de1_u = 1.0 / (cs + EPS)
    de2_u = 1.0 / (Gcs + EPS)
    dv_i = jax.lax.rsqrt(cs + Gcs + EPS)
    de1_i = 1.0 / (rs + EPS)
    de2_i = 1.0 / (Grs + EPS)

    U = u_ref[...]
    I = i_ref[...]
    W0 = W0_ref[...]
    b0 = b0_ref[...]
    W1 = W1_ref[...]
    b1 = b1_ref[...]

    v1 = _mm(HT, dv_u * U)
    r = _mm(H, jnp.concatenate([v1, dv_i * I], axis=1))
    v2, w1 = r[:, :D], r[:, D:]
    r = _mm(HT, jnp.concatenate([v2, w1], axis=1))
    v3, w2 = r[:, :D], r[:, D:]
    r = _mm(H, jnp.concatenate([de2_u * v3, w2], axis=1))
    v4, w3 = r[:, :D], r[:, D:]
    r = _mm(HT, jnp.concatenate([v4, de2_i * w3], axis=1))
    v5, w4 = r[:, :D], r[:, D:]
    r = _mm(H, jnp.concatenate([de1_u * v1 + v5, w4], axis=1))
    v6, w5 = r[:, :D], r[:, D:]
    U1 = _mm(dv_u * v6 + U, W0) + b0
    u1_ref[...] = U1
    r = _mm(HT, jnp.concatenate([dv_u * U1, de1_i * w1 + w5], axis=1))
    a2, w6 = r[:, :D], r[:, D:]
    I1 = _mm(dv_i * w6 + I, W0) + b0
    i1_ref[...] = I1
    r = _mm(H, jnp.concatenate([a2, dv_i * I1], axis=1))
    v2b, w1b = r[:, :D], r[:, D:]
    r = _mm(HT, jnp.concatenate([v2b, w1b], axis=1))
    v3b, w2b = r[:, :D], r[:, D:]
    r = _mm(H, jnp.concatenate([de2_u * v3b, w2b], axis=1))
    v4b, w3b = r[:, :D], r[:, D:]
    r = _mm(HT, jnp.concatenate([v4b, de2_i * w3b], axis=1))
    v5b, w4b = r[:, :D], r[:, D:]
    r = _mm(H, jnp.concatenate([de1_u * a2 + v5b, w4b], axis=1))
    v6b, w5b = r[:, :D], r[:, D:]
    u2_ref[...] = _mm(dv_u * v6b + U1, W1) + b1
    w6b = _mm(HT, de1_i * w1b + w5b)
    i2_ref[...] = _mm(dv_i * w6b + I1, W1) + b1


@functools.partial(jax.jit, static_argnames=("interpret",))
def _dhcf_tc(H, HT, user_emb, item_emb, W0, b0, W1, b1, interpret=False):
    out = jax.ShapeDtypeStruct((N_U, D), jnp.float32)
    return pl.pallas_call(
        _dhcf_body,
        out_shape=(out, out, out, out),
        interpret=interpret,
    )(H, HT, user_emb, item_emb, W0, b0.reshape(1, D), W1, b1.reshape(1, D))


def kernel(user_emb, item_emb, W0, b0, W1, b1, rows, cols):
    H = jnp.zeros((N_U, N_I), jnp.float32).at[rows, cols].add(1.0)
    HT = jnp.zeros((N_I, N_U), jnp.float32).at[cols, rows].add(1.0)
    u1, u2, i1, i2 = _dhcf_tc(H, HT, user_emb, item_emb, W0, b0, W1, b1)
    U_out = jnp.concatenate([user_emb, u1, u2], axis=1)
    I_out = jnp.concatenate([item_emb, i1, i2], axis=1)
    return (U_out, I_out)
